# baseline (speedup 1.0000x reference)
import functools

import jax
import jax.numpy as jnp
from jax import lax
from jax.experimental import pallas as pl
from jax.experimental.pallas import tpu as pltpu
from jax.experimental.pallas import tpu_sc as plsc

EPS = 1e-05
LANES = 16
CHUNK = 16
NBUF = 4


def _rsqrt_scalar_list(xs):
    halves = [0.5 * x for x in xs]
    ii = [lax.bitcast_convert_type(x, jnp.int32) for x in xs]
    ii = [jnp.int32(0x5F3759DF) - lax.shift_right_arithmetic(i, 1)
          for i in ii]
    ys = [lax.bitcast_convert_type(i, jnp.float32) for i in ii]
    for _ in range(2):
        qs = [y * y for y in ys]
        ps = [h * q for h, q in zip(halves, qs)]
        ss = [1.5 - p for p in ps]
        ys = [y * s for y, s in zip(ys, ss)]
    return ys


def _scalar_tree_sum(vals):
    while len(vals) > 1:
        vals = [vals[i] + vals[i + 1] for i in range(0, len(vals) - 1, 2)] \
            + ([vals[-1]] if len(vals) % 2 else [])
    return vals[0]


def kernel(input_ids, table, gamma, beta):
    B, S = input_ids.shape
    V, H = table.shape
    T = B * S
    n_slices = H // LANES

    mesh = plsc.VectorSubcoreMesh(core_axis_name="c", subcore_axis_name="s")
    NC, NS = mesh.num_cores, mesh.num_subcores
    NW = NC * NS
    tok_per_w = T // NW
    n_chunks = tok_per_w // CHUNK
    n_groups = n_chunks // NBUF
    assert T == NW * tok_per_w and tok_per_w == n_chunks * CHUNK
    assert n_chunks == n_groups * NBUF and n_groups >= 2

    def body(ids_hbm, table_hbm, gamma_hbm, beta_hbm, out_hbm,
             idx_v, b0, b1, b2, b3, ob0, ob1, gam_v, bet_v, stat_v,
             si0, si1, si2, si3, so0, so1):
        in_bufs = [b0, b1, b2, b3]
        out_bufs = [ob0, ob1]
        sin = [si0, si1, si2, si3]
        sout = [so0, so1]

        wid = lax.axis_index("c") * NS + lax.axis_index("s")
        base = wid * tok_per_w
        pltpu.sync_copy(ids_hbm.at[pl.ds(base, tok_per_w)], idx_v)
        pltpu.sync_copy(gamma_hbm, gam_v)
        pltpu.sync_copy(beta_hbm, bet_v)

        def idx_vec(c):
            return idx_v[pl.ds(c * CHUNK, CHUNK)]

        def start_in(c, b):
            pltpu.async_copy(table_hbm.at[idx_vec(c)], in_bufs[b], sin[b])

        def wait_in(c, b):
            pltpu.make_async_copy(table_hbm.at[idx_vec(c)], in_bufs[b],
                                  sin[b]).wait()

        def start_out(c, ob):
            pltpu.async_copy(out_bufs[ob],
                             out_hbm.at[pl.ds(base + c * CHUNK, CHUNK)],
                             sout[ob])

        def wait_out(ob):
            pltpu.make_async_copy(out_bufs[ob],
                                  out_hbm.at[pl.ds(base, CHUNK)],
                                  sout[ob]).wait()

        def compute(b, ob):
            src = in_bufs[b]
            dst = out_bufs[ob]
            grp = 8

            def rg_body(rg, carry):
                r0 = rg * grp
                rows = [r0 + k for k in range(grp)]
                accs = [jnp.zeros((LANES,), jnp.float32) for _ in range(grp)]
                sqs = [jnp.zeros((LANES,), jnp.float32) for _ in range(grp)]
                for j in range(n_slices):
                    sl = pl.ds(j * LANES, LANES)
                    vs = [src[r, sl] for r in rows]
                    qs = [v * v for v in vs]
                    accs = [a + v for a, v in zip(accs, vs)]
                    sqs = [s + q for s, q in zip(sqs, qs)]
                sums = [_scalar_tree_sum([accs[k][c] for c in range(LANES)])
                        for k in range(grp)]
                sums2 = [_scalar_tree_sum([sqs[k][c] for c in range(LANES)])
                         for k in range(grp)]
                means = [s * (1.0 / H) for s in sums]
                vars_ = [s2 * (1.0 / H) - m * m
                         for s2, m in zip(sums2, means)]
                scales = _rsqrt_scalar_list([v + EPS for v in vars_])
                negms = [-(m * sc) for m, sc in zip(means, scales)]
                for j in range(n_slices):
                    sl = pl.ds(j * LANES, LANES)
                    g = gam_v[sl]
                    bt = bet_v[sl]
                    vs = [src[r, sl] for r in rows]
                    ts = [v * sc + nm
                          for v, sc, nm in zip(vs, scales, negms)]
                    os_ = [t * g + bt for t in ts]
                    for r, o in zip(rows, os_):
                        dst[r, sl] = o
                return carry

            lax.fori_loop(0, CHUNK // grp, rg_body, 0)

        start_in(0, 0)
        start_in(1, 1)
        start_in(2, 2)

        def group(g, carry):
            for bslot in range(NBUF):
                c = g * NBUF + bslot
                ob = bslot % 2
                wait_in(c, bslot)
                if bslot < 2:
                    @pl.when(g >= 1)
                    def _():
                        wait_out(ob)
                else:
                    wait_out(ob)
                compute(bslot, ob)
                start_out(c, ob)
                w = (bslot + 3) % NBUF
                if bslot == 0:
                    start_in(c + 3, w)
                else:
                    @pl.when(g < n_groups - 1)
                    def _():
                        start_in(c + 3, w)
            return carry

        lax.fori_loop(0, n_groups, group, 0)
        for ob in range(2):
            wait_out(ob)

    f = pl.kernel(
        body,
        out_type=jax.ShapeDtypeStruct((T, H), jnp.float32),
        mesh=mesh,
        scratch_types=[
            pltpu.VMEM((tok_per_w,), jnp.int32),
            pltpu.VMEM((CHUNK, H), jnp.float32),
            pltpu.VMEM((CHUNK, H), jnp.float32),
            pltpu.VMEM((CHUNK, H), jnp.float32),
            pltpu.VMEM((CHUNK, H), jnp.float32),
            pltpu.VMEM((CHUNK, H), jnp.float32),
            pltpu.VMEM((CHUNK, H), jnp.float32),
            pltpu.VMEM((H,), jnp.float32),
            pltpu.VMEM((H,), jnp.float32),
            pltpu.VMEM((2 * CHUNK, LANES), jnp.float32),
            pltpu.SemaphoreType.DMA,
            pltpu.SemaphoreType.DMA,
            pltpu.SemaphoreType.DMA,
            pltpu.SemaphoreType.DMA,
            pltpu.SemaphoreType.DMA,
            pltpu.SemaphoreType.DMA,
        ],
    )
    ids_flat = input_ids.reshape(T).astype(jnp.int32)
    out = f(ids_flat, table, gamma, beta)
    return out.reshape(B, S, H)


# speedup vs baseline: 2.6792x; 2.6792x over previous
import functools

import jax
import jax.numpy as jnp
from jax import lax
from jax.experimental import pallas as pl
from jax.experimental.pallas import tpu as pltpu
from jax.experimental.pallas import tpu_sc as plsc

EPS = 1e-05
CHUNK = 16
NBUF = 4
LN_BLK = 512


def _sc_gather(ids_flat, table, T, H):
    mesh = plsc.VectorSubcoreMesh(core_axis_name="c", subcore_axis_name="s")
    NC, NS = mesh.num_cores, mesh.num_subcores
    NW = NC * NS
    tok_per_w = T // NW
    n_chunks = tok_per_w // CHUNK
    n_groups = n_chunks // NBUF
    assert T == NW * tok_per_w and tok_per_w == n_chunks * CHUNK
    assert n_chunks == n_groups * NBUF and n_groups >= 2

    def body(ids_hbm, table_hbm, out_hbm,
             idx_v, b0, b1, b2, b3, si0, si1, si2, si3, so0, so1, so2, so3):
        bufs = [b0, b1, b2, b3]
        sin = [si0, si1, si2, si3]
        sout = [so0, so1, so2, so3]

        wid = lax.axis_index("c") * NS + lax.axis_index("s")
        base = wid * tok_per_w
        pltpu.sync_copy(ids_hbm.at[pl.ds(base, tok_per_w)], idx_v)

        def idx_vec(c):
            return idx_v[pl.ds(c * CHUNK, CHUNK)]

        def start_in(c, b):
            pltpu.async_copy(table_hbm.at[idx_vec(c)], bufs[b], sin[b])

        def wait_in(c, b):
            pltpu.make_async_copy(table_hbm.at[idx_vec(c)], bufs[b],
                                  sin[b]).wait()

        def start_out(c, b):
            pltpu.async_copy(bufs[b],
                             out_hbm.at[pl.ds(base + c * CHUNK, CHUNK)],
                             sout[b])

        def wait_out(b):
            pltpu.make_async_copy(bufs[b], out_hbm.at[pl.ds(base, CHUNK)],
                                  sout[b]).wait()

        start_in(0, 0)
        start_in(1, 1)

        def group(g, carry):
            for bslot in range(NBUF):
                c = g * NBUF + bslot
                wait_in(c, bslot)
                start_out(c, bslot)
                w = (bslot + 2) % NBUF
                if bslot < 2:
                    @pl.when(g >= 1)
                    def _():
                        wait_out(w)
                    start_in(c + 2, w)
                else:
                    @pl.when(g < n_groups - 1)
                    def _():
                        wait_out(w)
                        start_in(c + 2, w)
            return carry

        lax.fori_loop(0, n_groups, group, 0)
        for b in range(NBUF):
            wait_out(b)

    f = pl.kernel(
        body,
        out_type=jax.ShapeDtypeStruct((T, H), jnp.float32),
        mesh=mesh,
        scratch_types=[
            pltpu.VMEM((tok_per_w,), jnp.int32),
            pltpu.VMEM((CHUNK, H), jnp.float32),
            pltpu.VMEM((CHUNK, H), jnp.float32),
            pltpu.VMEM((CHUNK, H), jnp.float32),
            pltpu.VMEM((CHUNK, H), jnp.float32),
            pltpu.SemaphoreType.DMA,
            pltpu.SemaphoreType.DMA,
            pltpu.SemaphoreType.DMA,
            pltpu.SemaphoreType.DMA,
            pltpu.SemaphoreType.DMA,
            pltpu.SemaphoreType.DMA,
            pltpu.SemaphoreType.DMA,
            pltpu.SemaphoreType.DMA,
        ],
    )
    return f(ids_flat, table)


def _ln_body(x_ref, g_ref, b_ref, o_ref):
    x = x_ref[...]
    m = jnp.mean(x, axis=-1, keepdims=True)
    xc = x - m
    var = jnp.mean(xc * xc, axis=-1, keepdims=True)
    o_ref[...] = (xc * lax.rsqrt(var + EPS)) * g_ref[...] + b_ref[...]


def _tc_layernorm(rows, gamma2d, beta2d, T, H):
    grid = (T // LN_BLK,)
    row_spec = pl.BlockSpec((LN_BLK, H), lambda i: (i, 0))
    gb_spec = pl.BlockSpec((1, H), lambda i: (0, 0))
    return pl.pallas_call(
        _ln_body,
        grid=grid,
        in_specs=[row_spec, gb_spec, gb_spec],
        out_specs=row_spec,
        out_shape=jax.ShapeDtypeStruct((T, H), jnp.float32),
        compiler_params=pltpu.CompilerParams(
            dimension_semantics=("arbitrary",),
        ),
    )(rows, gamma2d, beta2d)


def kernel(input_ids, table, gamma, beta):
    B, S = input_ids.shape
    V, H = table.shape
    T = B * S
    ids_flat = input_ids.reshape(T).astype(jnp.int32)
    rows = _sc_gather(ids_flat, table, T, H)
    out = _tc_layernorm(rows, gamma.reshape(1, H), beta.reshape(1, H), T, H)
    return out.reshape(B, S, H)
